# baseline (speedup 1.0000x reference)
import jax
import jax.numpy as jnp
from jax import lax
from jax.experimental import pallas as pl
from jax.experimental.pallas import tpu as pltpu

EMBED_DIM = 2048
NUM_EXPERTS = 16
TOP_K = 2

BLOCK_T = 2048
BLOCK_T2 = 4096


def _matmul_block(x_ref, w_ref, logits_ref):
    logits_ref[...] = jax.lax.dot_general(
        x_ref[...], w_ref[...],
        dimension_numbers=(((1,), (1,)), ((), ())),
        preferred_element_type=jnp.float32,
    )


def _top2_block(logits_ref, idx_ref, wgt_ref):
    logits = logits_ref[...]
    iota = lax.broadcasted_iota(jnp.int32, logits.shape, 1)
    m1 = jnp.max(logits, axis=-1, keepdims=True)
    i1 = jnp.min(jnp.where(logits == m1, iota, NUM_EXPERTS), axis=-1,
                 keepdims=True)
    masked = jnp.where(iota == i1, -jnp.inf, logits)
    m2 = jnp.max(masked, axis=-1, keepdims=True)
    i2 = jnp.min(jnp.where(masked == m2, iota, NUM_EXPERTS), axis=-1,
                 keepdims=True)
    w1 = jax.nn.sigmoid(m1 - m2)
    idx_ref[...] = jnp.concatenate([i1, i2], axis=-1)
    wgt_ref[...] = jnp.concatenate([w1, 1.0 - w1], axis=-1)


def kernel(x, W):
    n_tokens = x.shape[0]
    logits = pl.pallas_call(
        _matmul_block,
        grid=(n_tokens // BLOCK_T,),
        in_specs=[
            pl.BlockSpec((BLOCK_T, EMBED_DIM), lambda i: (i, 0)),
            pl.BlockSpec((NUM_EXPERTS, EMBED_DIM), lambda i: (0, 0)),
        ],
        out_specs=pl.BlockSpec((BLOCK_T, NUM_EXPERTS), lambda i: (i, 0)),
        out_shape=jax.ShapeDtypeStruct((n_tokens, NUM_EXPERTS), jnp.float32),
    )(x, W)

    idx, wgt = pl.pallas_call(
        _top2_block,
        grid=(n_tokens // BLOCK_T2,),
        in_specs=[pl.BlockSpec((BLOCK_T2, NUM_EXPERTS), lambda i: (i, 0))],
        out_specs=(
            pl.BlockSpec((BLOCK_T2, TOP_K), lambda i: (i, 0)),
            pl.BlockSpec((BLOCK_T2, TOP_K), lambda i: (i, 0)),
        ),
        out_shape=(
            jax.ShapeDtypeStruct((n_tokens, TOP_K), jnp.int32),
            jax.ShapeDtypeStruct((n_tokens, TOP_K), jnp.float32),
        ),
    )(logits)
    return (idx, wgt, logits)


# speedup vs baseline: 1.1525x; 1.1525x over previous
import jax
import jax.numpy as jnp
from jax import lax
from jax.experimental import pallas as pl
from jax.experimental.pallas import tpu as pltpu

EMBED_DIM = 2048
NUM_EXPERTS = 16
TOP_K = 2

BLOCK_T = 2048
NSPLIT = 8
SUB_T = BLOCK_T // NSPLIT


def _router_block(x_hbm, w_ref, idx_ref, wgt_ref, logits_ref, x_buf, sems):
    i = pl.program_id(0)
    nsteps = pl.num_programs(0)

    def start_copies(step, slot):
        for s in range(NSPLIT):
            pltpu.make_async_copy(
                x_hbm.at[pl.ds(step * BLOCK_T + s * SUB_T, SUB_T), :],
                x_buf.at[slot, pl.ds(s * SUB_T, SUB_T), :],
                sems.at[slot, s],
            ).start()

    def wait_copies(step, slot):
        for s in range(NSPLIT):
            pltpu.make_async_copy(
                x_hbm.at[pl.ds(step * BLOCK_T + s * SUB_T, SUB_T), :],
                x_buf.at[slot, pl.ds(s * SUB_T, SUB_T), :],
                sems.at[slot, s],
            ).wait()

    slot = lax.rem(i, 2)
    nxt = lax.rem(i + 1, 2)

    @pl.when(i == 0)
    def _first():
        start_copies(0, 0)

    @pl.when(i + 1 < nsteps)
    def _prefetch():
        start_copies(i + 1, nxt)

    wait_copies(i, slot)

    logits = jax.lax.dot_general(
        x_buf[slot], w_ref[...],
        dimension_numbers=(((1,), (1,)), ((), ())),
        preferred_element_type=jnp.float32,
    )
    logits_ref[...] = logits

    iota = lax.broadcasted_iota(jnp.int32, logits.shape, 1)
    m1 = jnp.max(logits, axis=-1, keepdims=True)
    i1 = jnp.min(jnp.where(logits == m1, iota, NUM_EXPERTS), axis=-1,
                 keepdims=True)
    masked = jnp.where(iota == i1, -jnp.inf, logits)
    m2 = jnp.max(masked, axis=-1, keepdims=True)
    i2 = jnp.min(jnp.where(masked == m2, iota, NUM_EXPERTS), axis=-1,
                 keepdims=True)
    w1 = jax.nn.sigmoid(m1 - m2)
    idx_ref[...] = jnp.concatenate([i1, i2], axis=-1)
    wgt_ref[...] = jnp.concatenate([w1, 1.0 - w1], axis=-1)


def kernel(x, W):
    n_tokens = x.shape[0]
    grid = (n_tokens // BLOCK_T,)
    out_types = (
        jax.ShapeDtypeStruct((n_tokens, TOP_K), jnp.int32),
        jax.ShapeDtypeStruct((n_tokens, TOP_K), jnp.float32),
        jax.ShapeDtypeStruct((n_tokens, NUM_EXPERTS), jnp.float32),
    )
    idx, wgt, logits = pl.pallas_call(
        _router_block,
        grid=grid,
        in_specs=[
            pl.BlockSpec(memory_space=pl.ANY),
            pl.BlockSpec((NUM_EXPERTS, EMBED_DIM), lambda i: (0, 0)),
        ],
        out_specs=(
            pl.BlockSpec((BLOCK_T, TOP_K), lambda i: (i, 0)),
            pl.BlockSpec((BLOCK_T, TOP_K), lambda i: (i, 0)),
            pl.BlockSpec((BLOCK_T, NUM_EXPERTS), lambda i: (i, 0)),
        ),
        out_shape=out_types,
        scratch_shapes=[
            pltpu.VMEM((2, BLOCK_T, EMBED_DIM), jnp.float32),
            pltpu.SemaphoreType.DMA((2, NSPLIT)),
        ],
    )(x, W)
    return (idx, wgt, logits)
